# baseline (speedup 1.0000x reference)
import functools

import jax
import jax.numpy as jnp
from jax import lax
from jax.experimental import pallas as pl
from jax.experimental.pallas import tpu as pltpu
from jax.experimental.pallas import tpu_sc as plsc

_K = 32
_EPS = 1e-5
_SLOPE = 0.01
_F32 = jnp.float32


def _prep_body(p2cat_ref, q1cat_ref, w0at_ref, w0bt_ref, a_ref, bq_ref):
    a_ref[...] = jnp.dot(p2cat_ref[...], w0at_ref[...],
                         preferred_element_type=_F32)
    bq_ref[...] = jnp.dot(q1cat_ref[...], w0bt_ref[...],
                          preferred_element_type=_F32)


def _knn_body(p1h_ref, p2ht_ref, gidx_ref, *, M, TN):
    b = pl.program_id(0)
    q = p1h_ref[...]
    pm = p2ht_ref[0]
    dx = q[:, 0:1] - pm[0:1, :]
    dy = q[:, 1:2] - pm[1:2, :]
    dz = q[:, 2:3] - pm[2:3, :]
    d = dx * dx + dy * dy + dz * dz
    miota = lax.broadcasted_iota(jnp.int32, (TN, M), 1)
    kiota = lax.broadcasted_iota(jnp.int32, (TN, _K), 1)

    def step(j, carry):
        d, acc = carry
        mn = jnp.min(d, axis=1, keepdims=True)
        idxj = jnp.min(jnp.where(d == mn, miota, M), axis=1, keepdims=True)
        acc = jnp.where(kiota == j, idxj, acc)
        d = jnp.where(miota == idxj, 3.0e38, d)
        return d, acc

    _, acc = lax.fori_loop(0, _K, step,
                           (d, jnp.zeros((TN, _K), jnp.int32)))
    gidx_ref[...] = acc + b * M


def _sc_gather(table, idx_flat, total, gwidth, owidth):
    nc, ns = 2, 16
    nw = nc * ns
    per_w = total // nw
    ch = 128
    n_ch = per_w // ch
    mesh = plsc.VectorSubcoreMesh(core_axis_name="c", subcore_axis_name="s")

    @functools.partial(
        pl.kernel, mesh=mesh,
        compiler_params=pltpu.CompilerParams(use_tc_tiling_on_sc=False),
        out_type=jax.ShapeDtypeStruct((total, owidth), _F32),
        scratch_types=[
            pltpu.VMEM((ch,), jnp.int32),
            pltpu.VMEM((ch, gwidth), _F32),
            pltpu.SemaphoreType.DMA,
        ],
    )
    def gk(table_hbm, idx_hbm, out_hbm, idx_v, rows_v, sem):
        wid = lax.axis_index("s") * nc + lax.axis_index("c")
        base = pl.multiple_of(wid * per_w, ch)

        def body(i, carry):
            off = pl.multiple_of(base + i * ch, ch)
            pltpu.sync_copy(idx_hbm.at[pl.ds(off, ch)], idx_v)
            pltpu.async_copy(table_hbm.at[idx_v], rows_v, sem).wait()
            pltpu.sync_copy(rows_v, out_hbm.at[pl.ds(off, ch)])
            return carry

        lax.fori_loop(0, n_ch, body, 0)

    return gk(table, idx_flat)


def _stats0_body(y0b_ref, bq_ref, sum_ref, sumsq_ref):
    i = pl.program_id(0)
    y0 = y0b_ref[...] + bq_ref[...]

    @pl.when(i == 0)
    def _():
        sum_ref[...] = jnp.zeros_like(sum_ref)
        sumsq_ref[...] = jnp.zeros_like(sumsq_ref)

    s = jnp.sum(y0, axis=0, keepdims=True)
    ss = jnp.sum(y0 * y0, axis=0, keepdims=True)
    sum_ref[...] += jnp.broadcast_to(s, sum_ref.shape)
    sumsq_ref[...] += jnp.broadcast_to(ss, sumsq_ref.shape)


def _mid_bq_body(y_ref, bq_ref, sc_ref, sh_ref, wt_ref,
                 out_ref, sum_ref, sumsq_ref):
    i = pl.program_id(0)
    y = y_ref[...] + bq_ref[...]
    z = y * sc_ref[0:1, :] + sh_ref[0:1, :]
    h = jnp.where(z >= 0, z, _SLOPE * z)
    o = jnp.dot(h, wt_ref[...], preferred_element_type=_F32)
    out_ref[...] = o

    @pl.when(i == 0)
    def _():
        sum_ref[...] = jnp.zeros_like(sum_ref)
        sumsq_ref[...] = jnp.zeros_like(sumsq_ref)

    sum_ref[...] += jnp.broadcast_to(jnp.sum(o, axis=0, keepdims=True),
                                     sum_ref.shape)
    sumsq_ref[...] += jnp.broadcast_to(jnp.sum(o * o, axis=0, keepdims=True),
                                       sumsq_ref.shape)


def _mid_max_body(y_ref, sc_ref, sh_ref, wt_ref,
                  mx_ref, mn_ref, sum_ref, sumsq_ref):
    qb = pl.program_id(0)
    k = pl.program_id(1)
    z = y_ref[...] * sc_ref[0:1, :] + sh_ref[0:1, :]
    h = jnp.where(z >= 0, z, _SLOPE * z)
    o = jnp.dot(h, wt_ref[...], preferred_element_type=_F32)

    @pl.when((qb == 0) & (k == 0))
    def _():
        sum_ref[...] = jnp.zeros_like(sum_ref)
        sumsq_ref[...] = jnp.zeros_like(sumsq_ref)

    sum_ref[...] += jnp.broadcast_to(jnp.sum(o, axis=0, keepdims=True),
                                     sum_ref.shape)
    sumsq_ref[...] += jnp.broadcast_to(jnp.sum(o * o, axis=0, keepdims=True),
                                       sumsq_ref.shape)

    @pl.when(k == 0)
    def _():
        mx_ref[...] = o
        mn_ref[...] = o

    @pl.when(k > 0)
    def _():
        mx_ref[...] = jnp.maximum(mx_ref[...], o)
        mn_ref[...] = jnp.minimum(mn_ref[...], o)


def _final_body(mx_ref, mn_ref, sc_ref, sh_ref, out_ref):
    sc = sc_ref[0:1, :]
    z = jnp.where(sc >= 0, mx_ref[...] * sc, mn_ref[...] * sc) + sh_ref[0:1, :]
    out_ref[...] = jnp.where(z >= 0, z, _SLOPE * z)


def _affine(g, b, s, ss, cnt):
    mu = s / cnt
    var = ss / cnt - mu * mu
    sc = g / jnp.sqrt(var + _EPS)
    sh = b - mu * sc
    rows = 8
    return (jnp.broadcast_to(sc[None, :], (rows, sc.shape[0])),
            jnp.broadcast_to(sh[None, :], (rows, sh.shape[0])))


def kernel(pos1, pos2, feature1, feature2, radius,
           W0, g0, b0, W1, g1, b1, W2, g2, b2):
    del radius
    B, _, N = pos1.shape
    M = pos2.shape[2]
    C = feature1.shape[1]
    BN, BM, BNK = B * N, B * M, B * N * _K
    O0, O1, O2 = W0.shape[0], W1.shape[0], W2.shape[0]

    p1t = pos1.transpose(0, 2, 1).reshape(BN, 3).astype(_F32)
    p2t = pos2.transpose(0, 2, 1).reshape(BM, 3).astype(_F32)
    f1t = feature1.transpose(0, 2, 1).reshape(BN, C).astype(_F32)
    f2t = feature2.transpose(0, 2, 1).reshape(BM, C).astype(_F32)
    p1pad = jnp.pad(p1t, ((0, 0), (0, 5)))
    p2pad = jnp.pad(p2t, ((0, 0), (0, 5)))
    p2cat = jnp.concatenate([p2t, f2t], axis=1)
    q1cat = jnp.concatenate([p1t, f1t], axis=1)
    w0at = W0[:, :3 + C].T.astype(_F32)
    w0bt = jnp.concatenate([-W0[:, :3], W0[:, 3 + C:]], axis=1).T.astype(_F32)

    TP = 2048
    npb = BM // TP
    row_spec = lambda w: pl.BlockSpec((TP, w), lambda i: (i, 0))
    full_spec = lambda r, w: pl.BlockSpec((r, w), lambda i: (0, 0))
    a, bq = pl.pallas_call(
        _prep_body,
        grid=(npb,),
        in_specs=[
            row_spec(3 + C), row_spec(3 + C),
            full_spec(3 + C, O0), full_spec(3 + C, O0),
        ],
        out_specs=(row_spec(O0), row_spec(O0)),
        out_shape=(
            jax.ShapeDtypeStruct((BM, O0), _F32),
            jax.ShapeDtypeStruct((BN, O0), _F32),
        ),
    )(p2cat, q1cat, w0at, w0bt)

    p1h = p1pad
    p2ht = p2pad.reshape(B, M, 8).transpose(0, 2, 1)

    TN = 1024
    nb = N // TN
    gidx = pl.pallas_call(
        functools.partial(_knn_body, M=M, TN=TN),
        grid=(B, nb),
        in_specs=[
            pl.BlockSpec((TN, 8), lambda b, i: (b * nb + i, 0)),
            pl.BlockSpec((1, 8, M), lambda b, i: (b, 0, 0)),
        ],
        out_specs=pl.BlockSpec((TN, _K), lambda b, i: (b * nb + i, 0)),
        out_shape=jax.ShapeDtypeStruct((BN, _K), jnp.int32),
    )(p1h, p2ht)

    idx_perm = gidx.T.reshape(BNK)
    y0b = _sc_gather(a, idx_perm, BNK, O0, O0)

    TS = 2048
    nblk = BNK // TS
    qbn = BN // TS

    acc_spec = pl.BlockSpec((8, O1), lambda i: (0, 0))
    s0, ss0 = pl.pallas_call(
        _stats0_body,
        grid=(nblk,),
        in_specs=[
            pl.BlockSpec((TS, O0), lambda i: (i, 0)),
            pl.BlockSpec((TS, O0), lambda i: (i % qbn, 0)),
        ],
        out_specs=(pl.BlockSpec((8, O0), lambda i: (0, 0)),
                   pl.BlockSpec((8, O0), lambda i: (0, 0))),
        out_shape=(jax.ShapeDtypeStruct((8, O0), _F32),
                   jax.ShapeDtypeStruct((8, O0), _F32)),
    )(y0b, bq)

    cnt = jnp.float32(BNK)
    sc0e, sh0e = _affine(g0, b0, s0[0], ss0[0], cnt)

    w1t = W1.T.astype(_F32)
    y1, s1, ss1 = pl.pallas_call(
        _mid_bq_body,
        grid=(nblk,),
        in_specs=[
            pl.BlockSpec((TS, O0), lambda i: (i, 0)),
            pl.BlockSpec((TS, O0), lambda i: (i % qbn, 0)),
            pl.BlockSpec((8, O0), lambda i: (0, 0)),
            pl.BlockSpec((8, O0), lambda i: (0, 0)),
            pl.BlockSpec((O0, O1), lambda i: (0, 0)),
        ],
        out_specs=(pl.BlockSpec((TS, O1), lambda i: (i, 0)),
                   acc_spec, acc_spec),
        out_shape=(jax.ShapeDtypeStruct((BNK, O1), _F32),
                   jax.ShapeDtypeStruct((8, O1), _F32),
                   jax.ShapeDtypeStruct((8, O1), _F32)),
    )(y0b, bq, sc0e, sh0e, w1t)

    sc1e, sh1e = _affine(g1, b1, s1[0], ss1[0], cnt)

    w2t = W2.T.astype(_F32)
    acc2_spec = pl.BlockSpec((8, O2), lambda qb, k: (0, 0))
    mx2, mn2, s2, ss2 = pl.pallas_call(
        _mid_max_body,
        grid=(qbn, _K),
        in_specs=[
            pl.BlockSpec((TS, O1), lambda qb, k: (k * qbn + qb, 0)),
            pl.BlockSpec((8, O1), lambda qb, k: (0, 0)),
            pl.BlockSpec((8, O1), lambda qb, k: (0, 0)),
            pl.BlockSpec((O1, O2), lambda qb, k: (0, 0)),
        ],
        out_specs=(pl.BlockSpec((TS, O2), lambda qb, k: (qb, 0)),
                   pl.BlockSpec((TS, O2), lambda qb, k: (qb, 0)),
                   acc2_spec, acc2_spec),
        out_shape=(jax.ShapeDtypeStruct((BN, O2), _F32),
                   jax.ShapeDtypeStruct((BN, O2), _F32),
                   jax.ShapeDtypeStruct((8, O2), _F32),
                   jax.ShapeDtypeStruct((8, O2), _F32)),
    )(y1, sc1e, sh1e, w2t)

    sc2e, sh2e = _affine(g2, b2, s2[0], ss2[0], cnt)

    o = pl.pallas_call(
        _final_body,
        grid=(qbn,),
        in_specs=[
            pl.BlockSpec((TS, O2), lambda qb: (qb, 0)),
            pl.BlockSpec((TS, O2), lambda qb: (qb, 0)),
            pl.BlockSpec((8, O2), lambda qb: (0, 0)),
            pl.BlockSpec((8, O2), lambda qb: (0, 0)),
        ],
        out_specs=pl.BlockSpec((TS, O2), lambda qb: (qb, 0)),
        out_shape=jax.ShapeDtypeStruct((BN, O2), _F32),
    )(mx2, mn2, sc2e, sh2e)

    feat1_new = o.reshape(B, N, O2).transpose(0, 2, 1)
    return (pos1, feat1_new)


# speedup vs baseline: 1.0317x; 1.0317x over previous
import functools

import jax
import jax.numpy as jnp
from jax import lax
from jax.experimental import pallas as pl
from jax.experimental.pallas import tpu as pltpu
from jax.experimental.pallas import tpu_sc as plsc

_K = 32
_EPS = 1e-5
_SLOPE = 0.01
_F32 = jnp.float32


def _prep_body(p2cat_ref, q1cat_ref, w0at_ref, w0bt_ref, a_ref, bq_ref):
    a_ref[...] = jnp.dot(p2cat_ref[...], w0at_ref[...],
                         preferred_element_type=_F32)
    bq_ref[...] = jnp.dot(q1cat_ref[...], w0bt_ref[...],
                          preferred_element_type=_F32)


def _knn_body(p1h_ref, p2ht_ref, gidx_ref, *, M, TN):
    b = pl.program_id(0)
    q = p1h_ref[...]
    pm = p2ht_ref[0]
    dx = q[:, 0:1] - pm[0:1, :]
    dy = q[:, 1:2] - pm[1:2, :]
    dz = q[:, 2:3] - pm[2:3, :]
    d = dx * dx + dy * dy + dz * dz
    miota = lax.broadcasted_iota(jnp.int32, (TN, M), 1)
    kiota = lax.broadcasted_iota(jnp.int32, (TN, _K), 1)

    def step(j, carry):
        d, acc = carry
        mn = jnp.min(d, axis=1, keepdims=True)
        idxj = jnp.min(jnp.where(d == mn, miota, M), axis=1, keepdims=True)
        acc = jnp.where(kiota == j, idxj, acc)
        d = jnp.where(miota == idxj, 3.0e38, d)
        return d, acc

    _, acc = lax.fori_loop(0, _K, step,
                           (d, jnp.zeros((TN, _K), jnp.int32)))
    gidx_ref[...] = acc + b * M


def _sc_gather(table, idx_flat, total, gwidth, owidth):
    nc, ns = 2, 16
    nw = nc * ns
    per_w = total // nw
    ch = 128
    n_ch = per_w // ch
    mesh = plsc.VectorSubcoreMesh(core_axis_name="c", subcore_axis_name="s")

    @functools.partial(
        pl.kernel, mesh=mesh,
        out_type=jax.ShapeDtypeStruct((total, owidth), _F32),
        scratch_types=[
            pltpu.VMEM((ch,), jnp.int32),
            pltpu.VMEM((ch, gwidth), _F32),
            pltpu.SemaphoreType.DMA,
        ],
    )
    def gk(table_hbm, idx_hbm, out_hbm, idx_v, rows_v, sem):
        wid = lax.axis_index("s") * nc + lax.axis_index("c")
        base = pl.multiple_of(wid * per_w, ch)

        def body(i, carry):
            off = pl.multiple_of(base + i * ch, ch)
            pltpu.sync_copy(idx_hbm.at[pl.ds(off, ch)], idx_v)
            pltpu.async_copy(table_hbm.at[idx_v], rows_v, sem).wait()
            pltpu.sync_copy(rows_v, out_hbm.at[pl.ds(off, ch)])
            return carry

        lax.fori_loop(0, n_ch, body, 0)

    return gk(table, idx_flat)


def _stats0_body(y0b_ref, bq_ref, sum_ref, sumsq_ref):
    i = pl.program_id(0)
    y0 = y0b_ref[...][:, :bq_ref.shape[1]] + bq_ref[...]

    @pl.when(i == 0)
    def _():
        sum_ref[...] = jnp.zeros_like(sum_ref)
        sumsq_ref[...] = jnp.zeros_like(sumsq_ref)

    s = jnp.sum(y0, axis=0, keepdims=True)
    ss = jnp.sum(y0 * y0, axis=0, keepdims=True)
    sum_ref[...] += jnp.broadcast_to(s, sum_ref.shape)
    sumsq_ref[...] += jnp.broadcast_to(ss, sumsq_ref.shape)


def _mid_bq_body(y_ref, bq_ref, sc_ref, sh_ref, wt_ref,
                 out_ref, sum_ref, sumsq_ref):
    i = pl.program_id(0)
    y = y_ref[...][:, :bq_ref.shape[1]] + bq_ref[...]
    z = y * sc_ref[0:1, :] + sh_ref[0:1, :]
    h = jnp.where(z >= 0, z, _SLOPE * z)
    o = jnp.dot(h, wt_ref[...], preferred_element_type=_F32)
    out_ref[...] = o

    @pl.when(i == 0)
    def _():
        sum_ref[...] = jnp.zeros_like(sum_ref)
        sumsq_ref[...] = jnp.zeros_like(sumsq_ref)

    sum_ref[...] += jnp.broadcast_to(jnp.sum(o, axis=0, keepdims=True),
                                     sum_ref.shape)
    sumsq_ref[...] += jnp.broadcast_to(jnp.sum(o * o, axis=0, keepdims=True),
                                       sumsq_ref.shape)


def _mid_max_body(y_ref, sc_ref, sh_ref, wt_ref,
                  mx_ref, mn_ref, sum_ref, sumsq_ref):
    qb = pl.program_id(0)
    k = pl.program_id(1)
    z = y_ref[...] * sc_ref[0:1, :] + sh_ref[0:1, :]
    h = jnp.where(z >= 0, z, _SLOPE * z)
    o = jnp.dot(h, wt_ref[...], preferred_element_type=_F32)

    @pl.when((qb == 0) & (k == 0))
    def _():
        sum_ref[...] = jnp.zeros_like(sum_ref)
        sumsq_ref[...] = jnp.zeros_like(sumsq_ref)

    sum_ref[...] += jnp.broadcast_to(jnp.sum(o, axis=0, keepdims=True),
                                     sum_ref.shape)
    sumsq_ref[...] += jnp.broadcast_to(jnp.sum(o * o, axis=0, keepdims=True),
                                       sumsq_ref.shape)

    @pl.when(k == 0)
    def _():
        mx_ref[...] = o
        mn_ref[...] = o

    @pl.when(k > 0)
    def _():
        mx_ref[...] = jnp.maximum(mx_ref[...], o)
        mn_ref[...] = jnp.minimum(mn_ref[...], o)


def _final_body(mx_ref, mn_ref, sc_ref, sh_ref, out_ref):
    sc = sc_ref[0:1, :]
    z = jnp.where(sc >= 0, mx_ref[...] * sc, mn_ref[...] * sc) + sh_ref[0:1, :]
    out_ref[...] = jnp.where(z >= 0, z, _SLOPE * z)


def _affine(g, b, s, ss, cnt):
    mu = s / cnt
    var = ss / cnt - mu * mu
    sc = g / jnp.sqrt(var + _EPS)
    sh = b - mu * sc
    rows = 8
    return (jnp.broadcast_to(sc[None, :], (rows, sc.shape[0])),
            jnp.broadcast_to(sh[None, :], (rows, sh.shape[0])))


def kernel(pos1, pos2, feature1, feature2, radius,
           W0, g0, b0, W1, g1, b1, W2, g2, b2):
    del radius
    B, _, N = pos1.shape
    M = pos2.shape[2]
    C = feature1.shape[1]
    BN, BM, BNK = B * N, B * M, B * N * _K
    O0, O1, O2 = W0.shape[0], W1.shape[0], W2.shape[0]

    p1t = pos1.transpose(0, 2, 1).reshape(BN, 3).astype(_F32)
    p2t = pos2.transpose(0, 2, 1).reshape(BM, 3).astype(_F32)
    f1t = feature1.transpose(0, 2, 1).reshape(BN, C).astype(_F32)
    f2t = feature2.transpose(0, 2, 1).reshape(BM, C).astype(_F32)
    p1pad = jnp.pad(p1t, ((0, 0), (0, 5)))
    p2pad = jnp.pad(p2t, ((0, 0), (0, 5)))
    p2cat = jnp.concatenate([p2t, f2t], axis=1)
    q1cat = jnp.concatenate([p1t, f1t], axis=1)
    w0at = jnp.pad(W0[:, :3 + C].T.astype(_F32), ((0, 0), (0, 128 - O0)))
    w0bt = jnp.concatenate([-W0[:, :3], W0[:, 3 + C:]], axis=1).T.astype(_F32)

    TP = 2048
    npb = BM // TP
    row_spec = lambda w: pl.BlockSpec((TP, w), lambda i: (i, 0))
    full_spec = lambda r, w: pl.BlockSpec((r, w), lambda i: (0, 0))
    a, bq = pl.pallas_call(
        _prep_body,
        grid=(npb,),
        in_specs=[
            row_spec(3 + C), row_spec(3 + C),
            full_spec(3 + C, 128), full_spec(3 + C, O0),
        ],
        out_specs=(row_spec(128), row_spec(O0)),
        out_shape=(
            jax.ShapeDtypeStruct((BM, 128), _F32),
            jax.ShapeDtypeStruct((BN, O0), _F32),
        ),
    )(p2cat, q1cat, w0at, w0bt)

    p1h = p1pad
    p2ht = p2pad.reshape(B, M, 8).transpose(0, 2, 1)

    TN = 1024
    nb = N // TN
    gidx = pl.pallas_call(
        functools.partial(_knn_body, M=M, TN=TN),
        grid=(B, nb),
        in_specs=[
            pl.BlockSpec((TN, 8), lambda b, i: (b * nb + i, 0)),
            pl.BlockSpec((1, 8, M), lambda b, i: (b, 0, 0)),
        ],
        out_specs=pl.BlockSpec((TN, _K), lambda b, i: (b * nb + i, 0)),
        out_shape=jax.ShapeDtypeStruct((BN, _K), jnp.int32),
    )(p1h, p2ht)

    idx_perm = gidx.T.reshape(BNK)
    y0b = _sc_gather(a, idx_perm, BNK, 128, 128)

    TS = 2048
    nblk = BNK // TS
    qbn = BN // TS

    acc_spec = pl.BlockSpec((8, O1), lambda i: (0, 0))
    s0, ss0 = pl.pallas_call(
        _stats0_body,
        grid=(nblk,),
        in_specs=[
            pl.BlockSpec((TS, 128), lambda i: (i, 0)),
            pl.BlockSpec((TS, O0), lambda i: (i % qbn, 0)),
        ],
        out_specs=(pl.BlockSpec((8, O0), lambda i: (0, 0)),
                   pl.BlockSpec((8, O0), lambda i: (0, 0))),
        out_shape=(jax.ShapeDtypeStruct((8, O0), _F32),
                   jax.ShapeDtypeStruct((8, O0), _F32)),
    )(y0b, bq)

    cnt = jnp.float32(BNK)
    sc0e, sh0e = _affine(g0, b0, s0[0], ss0[0], cnt)

    w1t = W1.T.astype(_F32)
    y1, s1, ss1 = pl.pallas_call(
        _mid_bq_body,
        grid=(nblk,),
        in_specs=[
            pl.BlockSpec((TS, 128), lambda i: (i, 0)),
            pl.BlockSpec((TS, O0), lambda i: (i % qbn, 0)),
            pl.BlockSpec((8, O0), lambda i: (0, 0)),
            pl.BlockSpec((8, O0), lambda i: (0, 0)),
            pl.BlockSpec((O0, O1), lambda i: (0, 0)),
        ],
        out_specs=(pl.BlockSpec((TS, O1), lambda i: (i, 0)),
                   acc_spec, acc_spec),
        out_shape=(jax.ShapeDtypeStruct((BNK, O1), _F32),
                   jax.ShapeDtypeStruct((8, O1), _F32),
                   jax.ShapeDtypeStruct((8, O1), _F32)),
    )(y0b, bq, sc0e, sh0e, w1t)

    sc1e, sh1e = _affine(g1, b1, s1[0], ss1[0], cnt)

    w2t = W2.T.astype(_F32)
    acc2_spec = pl.BlockSpec((8, O2), lambda qb, k: (0, 0))
    mx2, mn2, s2, ss2 = pl.pallas_call(
        _mid_max_body,
        grid=(qbn, _K),
        in_specs=[
            pl.BlockSpec((TS, O1), lambda qb, k: (k * qbn + qb, 0)),
            pl.BlockSpec((8, O1), lambda qb, k: (0, 0)),
            pl.BlockSpec((8, O1), lambda qb, k: (0, 0)),
            pl.BlockSpec((O1, O2), lambda qb, k: (0, 0)),
        ],
        out_specs=(pl.BlockSpec((TS, O2), lambda qb, k: (qb, 0)),
                   pl.BlockSpec((TS, O2), lambda qb, k: (qb, 0)),
                   acc2_spec, acc2_spec),
        out_shape=(jax.ShapeDtypeStruct((BN, O2), _F32),
                   jax.ShapeDtypeStruct((BN, O2), _F32),
                   jax.ShapeDtypeStruct((8, O2), _F32),
                   jax.ShapeDtypeStruct((8, O2), _F32)),
    )(y1, sc1e, sh1e, w2t)

    sc2e, sh2e = _affine(g2, b2, s2[0], ss2[0], cnt)

    o = pl.pallas_call(
        _final_body,
        grid=(qbn,),
        in_specs=[
            pl.BlockSpec((TS, O2), lambda qb: (qb, 0)),
            pl.BlockSpec((TS, O2), lambda qb: (qb, 0)),
            pl.BlockSpec((8, O2), lambda qb: (0, 0)),
            pl.BlockSpec((8, O2), lambda qb: (0, 0)),
        ],
        out_specs=pl.BlockSpec((TS, O2), lambda qb: (qb, 0)),
        out_shape=jax.ShapeDtypeStruct((BN, O2), _F32),
    )(mx2, mn2, sc2e, sh2e)

    feat1_new = o.reshape(B, N, O2).transpose(0, 2, 1)
    return (pos1, feat1_new)
